# baseline (speedup 1.0000x reference)
import functools

import jax
import jax.numpy as jnp
from jax import lax
from jax.experimental import pallas as pl
from jax.experimental.pallas import tpu as pltpu
from jax.experimental.pallas import tpu_sc as plsc

_MAX_LEN = 512
_LANES = 16


def _positional_encodings(max_len, embed_dim):
    pos = jnp.arange(0, max_len, dtype=jnp.float32).reshape(-1, 1)
    skip = jnp.arange(0, embed_dim, 2, dtype=jnp.float32)
    denom = 10000.0 ** (skip / embed_dim)
    enc = jnp.zeros((max_len, embed_dim), dtype=jnp.float32)
    enc = enc.at[:, 0::2].set(jnp.sin(pos / denom))
    enc = enc.at[:, 1::2].set(jnp.cos(pos / denom))
    return enc


def _repack_table(tbl_t, tailf, V, D, NC, NW):
    VC = 256
    n_full = V // VC
    tail = V - n_full * VC
    n_units = n_full + (1 if tail else 0)
    fpu = VC * D
    n_k = (n_units + NW - 1) // NW

    mesh = plsc.VectorSubcoreMesh(core_axis_name="c", subcore_axis_name="s")

    @functools.partial(
        pl.kernel,
        mesh=mesh,
        compiler_params=pltpu.CompilerParams(
            use_tc_tiling_on_sc=True, needs_layout_passes=False),
        out_type=jax.ShapeDtypeStruct((V * D,), jnp.float32),
        scratch_types=[
            pltpu.VMEM((D, VC), jnp.float32),
            pltpu.VMEM((D, VC), jnp.float32),
            pltpu.VMEM((fpu,), jnp.float32),
            pltpu.VMEM((fpu,), jnp.float32),
            pltpu.VMEM((tail * D if tail else 8,), jnp.float32),
            [pltpu.SemaphoreType.DMA] * 2,
            [pltpu.SemaphoreType.DMA] * 2,
        ],
    )
    def repack(tbl_hbm, tail_hbm, out_hbm, ib0, ib1, ob0, ob1, tb,
               isems, osems):
        ibs, obs = (ib0, ib1), (ob0, ob1)
        wid = lax.axis_index("s") * NC + lax.axis_index("c")
        iot = lax.iota(jnp.int32, _LANES)
        base0 = (iot >> 1) * 128 + (iot & 1) * 64

        def u_of(k):
            return wid + k * NW

        def issue_in(k, b):
            pltpu.async_copy(tbl_hbm.at[:, pl.ds(u_of(k) * VC, VC)],
                             ibs[b], isems[b])

        def wait_in(b):
            pltpu.make_async_copy(tbl_hbm.at[:, pl.ds(0, VC)],
                                  ibs[b], isems[b]).wait()

        def wait_out(b):
            pltpu.make_async_copy(obs[b], out_hbm.at[pl.ds(0, fpu)],
                                  osems[b]).wait()

        def transpose(b):
            @plsc.parallel_loop(0, VC // _LANES)
            def _(j):
                base_j = base0 + j * (_LANES // 2) * 128
                for d0 in range(0, D, 8):
                    vals = [ibs[b][d0 + i, pl.ds(j * _LANES, _LANES)]
                            for i in range(8)]
                    for i in range(8):
                        plsc.store_scatter(obs[b], [base_j + (d0 + i)],
                                           vals[i])

        def kbody(kk, carry):
            for b in range(2):
                k = kk * 2 + b
                u = u_of(k)

                @pl.when(u < n_full)
                def _():
                    @pl.when(u_of(k + 1) < n_full)
                    def _():
                        issue_in(k + 1, 1 - b)

                    wait_in(b)
                    transpose(b)

                    @pl.when(k >= 2)
                    def _():
                        wait_out(b)

                    pltpu.async_copy(
                        obs[b], out_hbm.at[pl.ds(u * fpu, fpu)],
                        osems[b])
            return carry

        @pl.when(u_of(0) < n_full)
        def _():
            issue_in(0, 0)

        lax.fori_loop(0, (n_k + 1) // 2, kbody, 0)

        wait_out(0)
        wait_out(1)

        if tail:
            @pl.when(wid == (n_full % NW))
            def _():
                pltpu.sync_copy(tail_hbm, tb)
                pltpu.sync_copy(tb,
                                out_hbm.at[pl.ds(n_full * fpu, tail * D)])

    return repack(tbl_t, tailf)


def _gather_emb(packed, ids_t, enc_flat, B, L, D, NC, NW):
    NB = 256
    n_units = L * (B // NB)
    units_w = n_units // NW
    nbl = B // NB
    NJ = NB // _LANES

    mesh = plsc.VectorSubcoreMesh(core_axis_name="c", subcore_axis_name="s")

    @functools.partial(
        pl.kernel,
        mesh=mesh,
        compiler_params=pltpu.CompilerParams(
            use_tc_tiling_on_sc=True, needs_layout_passes=False),
        out_type=jax.ShapeDtypeStruct((L, D, B), jnp.float32),
        scratch_types=[
            pltpu.VMEM((NB,), jnp.int32),
            pltpu.VMEM((NB,), jnp.int32),
            pltpu.VMEM((NB,), jnp.int32),
            pltpu.VMEM((NB,), jnp.int32),
            pltpu.VMEM((NB,), jnp.int32),
            pltpu.VMEM((NB,), jnp.int32),
            pltpu.VMEM((NB, 128), jnp.float32),
            pltpu.VMEM((NB, 128), jnp.float32),
            pltpu.VMEM((D, NB), jnp.float32),
            pltpu.VMEM((D, NB), jnp.float32),
            pltpu.VMEM((L * D,), jnp.float32),
            [pltpu.SemaphoreType.DMA] * 2,
            [pltpu.SemaphoreType.DMA] * 2,
            [pltpu.SemaphoreType.DMA] * 2,
        ],
    )
    def gather(tab_hbm, ids_hbm, enc_hbm, out_hbm, idxb0, idxb1, qb0, qb1,
               hb0, hb1, gb0, gb1, ob0, ob1, enc_v, isems, gsems, osems):
        idxbs, qbs, hbs = (idxb0, idxb1), (qb0, qb1), (hb0, hb1)
        gbs, obs = (gb0, gb1), (ob0, ob1)
        wid = lax.axis_index("s") * NC + lax.axis_index("c")
        iot = lax.iota(jnp.int32, _LANES)
        pltpu.sync_copy(enc_hbm, enc_v)

        def lb0(k):
            t = wid * units_w + k
            return t // nbl, (t % nbl) * NB

        def issue_idx(k, b):
            l, b0 = lb0(k)
            pltpu.async_copy(ids_hbm.at[l, pl.ds(b0, NB)], idxbs[b],
                             isems[b])

        def stage1(k, b):
            pltpu.make_async_copy(ids_hbm.at[0, pl.ds(0, NB)],
                                  idxbs[b], isems[b]).wait()

            @plsc.parallel_loop(0, NJ)
            def _(j):
                v = idxbs[b][pl.ds(j * _LANES, _LANES)]
                qbs[b][pl.ds(j * _LANES, _LANES)] = v >> 1
                hbs[b][pl.ds(j * _LANES, _LANES)] = (v & 1) * 64

            for o in range(0, NB, 128):
                pltpu.async_copy(tab_hbm.at[qbs[b].at[pl.ds(o, 128)]],
                                 gbs[b].at[pl.ds(o, 128)], gsems[b])

        def stage2(k, b):
            l, b0 = lb0(k)
            for o in range(0, NB, 128):
                pltpu.make_async_copy(tab_hbm.at[pl.ds(0, 128)],
                                      gbs[b].at[pl.ds(o, 128)],
                                      gsems[b]).wait()

            @pl.when(k >= 2)
            def _():
                pltpu.make_async_copy(obs[b],
                                      out_hbm.at[0, :, pl.ds(0, NB)],
                                      osems[b]).wait()

            gbb = gbs[b]
            obb = obs[b]
            hbb = hbs[b]

            @plsc.parallel_loop(0, D)
            def _(d):
                e = plsc.load_gather(
                    enc_v, [jnp.full((_LANES,), l * D + d, jnp.int32)])
                for j0 in range(0, NJ, 4):
                    cols = [hbb[pl.ds((j0 + j) * _LANES, _LANES)] + d
                            for j in range(4)]
                    vals = [plsc.load_gather(
                                gbb, [iot + (j0 + j) * _LANES, cols[j]])
                            for j in range(4)]
                    for j in range(4):
                        obb[d, pl.ds((j0 + j) * _LANES, _LANES)] = (
                            vals[j] + e)

            pltpu.async_copy(obb, out_hbm.at[l, :, pl.ds(b0, NB)], osems[b])

        issue_idx(0, 0)

        def kbody(kk, carry):
            for b in range(2):
                k = kk * 2 + b

                @pl.when(k + 1 < units_w)
                def _():
                    issue_idx(k + 1, 1 - b)

                stage1(k, b)

                @pl.when(k >= 1)
                def _():
                    stage2(k - 1, 1 - b)
            return carry

        lax.fori_loop(0, units_w // 2, kbody, 0)
        stage2(units_w - 1, 1)

        for b in range(2):
            pltpu.make_async_copy(obs[b], out_hbm.at[0, :, pl.ds(0, NB)],
                                  osems[b]).wait()

    return gather(packed, ids_t, enc_flat)


def kernel(input_ids, src_table):
    B, L = input_ids.shape
    V, D = src_table.shape
    info = plsc.get_sparse_core_info()
    NC, NS = info.num_cores, info.num_subcores
    NW = NC * NS
    assert D == 64 and B % 512 == 0
    assert (L * B // 256) % (2 * NW) == 0

    ids_t = input_ids.astype(jnp.int32).T
    tbl_t = src_table.T
    n_main = (V // 256) * 256
    tailf = src_table[n_main:].reshape(-1)
    enc_flat = _positional_encodings(_MAX_LEN, D)[:L].astype(
        jnp.float32).reshape(-1)

    packed = _repack_table(tbl_t, tailf, V, D, NC, NW).reshape(V // 2, 2 * D)
    out2 = _gather_emb(packed, ids_t, enc_flat, B, L, D, NC, NW)
    return out2.transpose(2, 0, 1)


# speedup vs baseline: 1.5370x; 1.5370x over previous
import functools

import jax
import jax.numpy as jnp
from jax import lax
from jax.experimental import pallas as pl
from jax.experimental.pallas import tpu as pltpu
from jax.experimental.pallas import tpu_sc as plsc

_MAX_LEN = 512
_LANES = 16
_NBUF = 4


def _positional_encodings(max_len, embed_dim):
    pos = jnp.arange(0, max_len, dtype=jnp.float32).reshape(-1, 1)
    skip = jnp.arange(0, embed_dim, 2, dtype=jnp.float32)
    denom = 10000.0 ** (skip / embed_dim)
    enc = jnp.zeros((max_len, embed_dim), dtype=jnp.float32)
    enc = enc.at[:, 0::2].set(jnp.sin(pos / denom))
    enc = enc.at[:, 1::2].set(jnp.cos(pos / denom))
    return enc


def kernel(input_ids, src_table):
    B, L = input_ids.shape
    V, D = src_table.shape
    ids = input_ids.astype(jnp.int32)
    enc = _positional_encodings(_MAX_LEN, D)[:L].astype(jnp.float32)

    info = plsc.get_sparse_core_info()
    NC, NS = info.num_cores, info.num_subcores
    NW = NC * NS
    assert B % NW == 0, (B, NW)
    rows_per_w = B // NW
    assert rows_per_w % _NBUF == 0
    assert D % _LANES == 0
    splits = [(o, min(128, L - o)) for o in range(0, L, 128)]

    mesh = plsc.VectorSubcoreMesh(core_axis_name="c", subcore_axis_name="s")

    @functools.partial(
        pl.kernel,
        mesh=mesh,
        compiler_params=pltpu.CompilerParams(use_tc_tiling_on_sc=False),
        out_type=jax.ShapeDtypeStruct((B, L, D), jnp.float32),
        scratch_types=[
            pltpu.VMEM((rows_per_w, L), jnp.int32),
            pltpu.VMEM((_NBUF, L, D), jnp.float32),
            pltpu.VMEM((L, D), jnp.float32),
            [pltpu.SemaphoreType.DMA] * _NBUF,
            [pltpu.SemaphoreType.DMA] * _NBUF,
        ],
    )
    def run(ids_hbm, table_hbm, enc_hbm, out_hbm, idx_all, rows_v, enc_v,
            gsems, osems):
        wid = lax.axis_index("s") * NC + lax.axis_index("c")
        row0 = wid * rows_per_w

        pltpu.sync_copy(enc_hbm, enc_v)
        pltpu.sync_copy(ids_hbm.at[pl.ds(row0, rows_per_w)], idx_all)

        def start_gather(g, b):
            for (o, n) in splits:
                pltpu.async_copy(
                    table_hbm.at[idx_all.at[g, pl.ds(o, n)]],
                    rows_v.at[b, pl.ds(o, n)],
                    gsems[b],
                )

        def wait_gather(b):
            pltpu.make_async_copy(
                table_hbm.at[pl.ds(0, L)], rows_v.at[b], gsems[b]).wait()

        def wait_out(b):
            pltpu.make_async_copy(
                rows_v.at[b], out_hbm.at[row0], osems[b]).wait()

        def add_enc(b):
            def add_body(i, carry):
                i0 = i * 4
                for dr in range(4):
                    for k in range(D // _LANES):
                        sl = pl.ds(k * _LANES, _LANES)
                        rows_v[b, i0 + dr, sl] = (
                            rows_v[b, i0 + dr, sl] + enc_v[i0 + dr, sl])
                return carry

            lax.fori_loop(0, L // 4, add_body, 0)
            rem = L % 4
            for dr in range(rem):
                for k in range(D // _LANES):
                    sl = pl.ds(k * _LANES, _LANES)
                    rows_v[b, L - rem + dr, sl] = (
                        rows_v[b, L - rem + dr, sl] + enc_v[L - rem + dr, sl])

        for b in range(_NBUF):
            start_gather(b, b)

        def outer(t, carry):
            for b in range(_NBUF):
                g = t * _NBUF + b
                bp = (b - 1) % _NBUF
                g_next = g + _NBUF - 1
                wait_gather(b)
                add_enc(b)

                @pl.when(jnp.logical_and(g_next >= _NBUF,
                                         g_next < rows_per_w))
                def _():
                    wait_out(bp)
                    start_gather(g_next, bp)

                pltpu.async_copy(rows_v.at[b], out_hbm.at[row0 + g], osems[b])
            return carry

        lax.fori_loop(0, rows_per_w // _NBUF, outer, 0)

        for b in range(_NBUF):
            wait_out(b)

    return run(ids, src_table, enc)
